# Initial kernel scaffold; baseline (speedup 1.0000x reference)
import jax
import jax.numpy as jnp
from jax.experimental import pallas as pl


def kernel(x, norm_w, router_w, router_b, w_gate_up, b_gate_up, w_down, b_down):
    raise NotImplementedError("write your pallas kernel here")


# speedup vs baseline: 3.5321x; 3.5321x over previous
import functools

import jax
import jax.numpy as jnp
from jax.experimental import pallas as pl
from jax.experimental.pallas import tpu as pltpu

T, D, F, E, TOP_K = 2048, 1024, 1024, 8, 2
LIMIT = 7.0
ALPHA = 1.702
EPS = 1e-5

BT = 1024


def _fused_body(x_ref, nw_ref, rwt_ref, rb_ref, wg_ref, wu_ref, wd_ref,
                bg_ref, bu_ref, bd_ref, out_ref, t_bf, cw_ref):
    e = pl.program_id(1)

    @pl.when(e == 0)
    def _prologue():
        xx = x_ref[...]
        ms = jnp.mean(xx * xx, axis=-1, keepdims=True)
        t = xx * jax.lax.rsqrt(ms + EPS) * nw_ref[...]
        t_bf[...] = t.astype(jnp.bfloat16)
        g = jnp.dot(t, rwt_ref[...], preferred_element_type=jnp.float32)
        g = g + rb_ref[...]
        m = jnp.max(g, axis=-1, keepdims=True)
        eg = jnp.exp(g - m)
        p = eg / jnp.sum(eg, axis=-1, keepdims=True)
        eidx = jax.lax.broadcasted_iota(jnp.int32, p.shape, 1)
        v1 = jnp.max(p, axis=-1, keepdims=True)
        i1 = jnp.min(jnp.where(p >= v1, eidx, E), axis=-1, keepdims=True)
        p2 = jnp.where(eidx == i1, -jnp.inf, p)
        v2 = jnp.max(p2, axis=-1, keepdims=True)
        i2 = jnp.min(jnp.where(p2 >= v2, eidx, E), axis=-1, keepdims=True)
        s = v1 + v2
        cw_ref[...] = (jnp.where(eidx == i1, v1 / s, 0.0)
                       + jnp.where(eidx == i2, v2 / s, 0.0))
        out_ref[...] = xx

    t = t_bf[...]
    gate = jnp.dot(t, wg_ref[0], preferred_element_type=jnp.float32)
    gate = gate + bg_ref[0]
    up = jnp.dot(t, wu_ref[0], preferred_element_type=jnp.float32)
    up = up + bu_ref[0]
    gate = jnp.minimum(gate, LIMIT)
    up = jnp.clip(up, -LIMIT, LIMIT)
    glu = gate * jax.nn.sigmoid(gate * ALPHA)
    act = ((up + 1.0) * glu).astype(jnp.bfloat16)
    y = jnp.dot(act, wd_ref[0], preferred_element_type=jnp.float32)
    y = y + bd_ref[0]
    cw = cw_ref[...]
    lane = jax.lax.broadcasted_iota(jnp.int32, cw.shape, 1)
    we = jnp.sum(jnp.where(lane == e, cw, 0.0), axis=-1, keepdims=True)
    out_ref[...] += we * y


@jax.jit
def _mlpblock(x, norm_w, router_w, router_b, w_gate_up, b_gate_up, w_down,
              b_down):
    rwt = router_w.T
    rb = router_b.reshape(1, E)
    wg = w_gate_up[:, :, 0::2].astype(jnp.bfloat16)
    wu = w_gate_up[:, :, 1::2].astype(jnp.bfloat16)
    bg = b_gate_up[:, 0::2].reshape(E, 1, F)
    bu = b_gate_up[:, 1::2].reshape(E, 1, F)
    wd = w_down.astype(jnp.bfloat16)
    bd = b_down.reshape(E, 1, D)
    nw = norm_w.reshape(1, D)

    grid = (T // BT, E)
    out = pl.pallas_call(
        _fused_body,
        grid=grid,
        in_specs=[
            pl.BlockSpec((BT, D), lambda i, e: (i, 0)),
            pl.BlockSpec((1, D), lambda i, e: (0, 0)),
            pl.BlockSpec((D, E), lambda i, e: (0, 0)),
            pl.BlockSpec((1, E), lambda i, e: (0, 0)),
            pl.BlockSpec((1, D, F), lambda i, e: (e, 0, 0)),
            pl.BlockSpec((1, D, F), lambda i, e: (e, 0, 0)),
            pl.BlockSpec((1, F, D), lambda i, e: (e, 0, 0)),
            pl.BlockSpec((1, 1, F), lambda i, e: (e, 0, 0)),
            pl.BlockSpec((1, 1, F), lambda i, e: (e, 0, 0)),
            pl.BlockSpec((1, 1, D), lambda i, e: (e, 0, 0)),
        ],
        out_specs=pl.BlockSpec((BT, D), lambda i, e: (i, 0)),
        out_shape=jax.ShapeDtypeStruct((T, D), jnp.float32),
        scratch_shapes=[
            pltpu.VMEM((BT, D), jnp.bfloat16),
            pltpu.VMEM((BT, E), jnp.float32),
        ],
        compiler_params=pltpu.CompilerParams(
            dimension_semantics=("arbitrary", "arbitrary"),
        ),
    )(x, nw, rwt, rb, wg, wu, wd, bg, bu, bd)
    return out


def kernel(x, norm_w, router_w, router_b, w_gate_up, b_gate_up, w_down,
           b_down):
    return _mlpblock(x, norm_w, router_w, router_b, w_gate_up, b_gate_up,
                     w_down, b_down)
